# Initial kernel scaffold; baseline (speedup 1.0000x reference)
import jax
import jax.numpy as jnp
from jax.experimental import pallas as pl


def kernel(x, edge_index, batch, W_enc, b_enc, Ws, bs, ps, W1, b1, W2, b2, W3, b3):
    raise NotImplementedError("write your pallas kernel here")


# speedup vs baseline: 7.1339x; 7.1339x over previous
import functools
import math

import jax
import jax.numpy as jnp
import numpy as np
from jax import lax
from jax.experimental import pallas as pl
from jax.experimental.pallas import tpu as pltpu
from jax.experimental.pallas import tpu_sc as plsc

NHID = 128
HEADS = 2
LAYERS = 2
G = 64
NTASKS = 10
N_NODES = 10000
N_EDGES = 320000

NPAD = 10112
R2D = NPAD // 128
EPAD = 327680
NWORK = 32
NSUB = 16
CHUNKS = EPAD // (NWORK * 128)
CH = 128
STRIPE = NPAD // NSUB
_I32MIN = np.int32(-(2 ** 31))


def _edge_agg_body(x_hbm, src_hbm, dst_hbm, zero_hbm, out_hbm,
                   src_v, dst_v, rows_v, acc_sh, sem):
    c = lax.axis_index("c")
    s = lax.axis_index("s")
    w = c * NSUB + s
    pltpu.sync_copy(zero_hbm, acc_sh.at[pl.ds(s * STRIPE, STRIPE)])
    pltpu.sync_copy(src_hbm.at[w], src_v)
    pltpu.sync_copy(dst_hbm.at[w], dst_v)
    plsc.subcore_barrier()

    def body(j, carry):
        pltpu.async_copy(x_hbm.at[src_v.at[j]], rows_v, sem).wait()
        pltpu.sync_copy(rows_v, acc_sh.at[dst_v.at[j]], add=True)
        return carry

    lax.fori_loop(0, CHUNKS, body, 0)
    plsc.subcore_barrier()
    pltpu.sync_copy(acc_sh.at[pl.ds(s * STRIPE, STRIPE)],
                    out_hbm.at[c, pl.ds(s * STRIPE, STRIPE)])


@functools.cache
def _edge_agg_kernel():
    return pl.kernel(
        _edge_agg_body,
        mesh=plsc.VectorSubcoreMesh(core_axis_name="c", subcore_axis_name="s"),
        out_type=jax.ShapeDtypeStruct((2, NPAD, NHID), jnp.float32),
        scratch_types=[
            pltpu.VMEM((CHUNKS, CH), jnp.int32),
            pltpu.VMEM((CHUNKS, CH), jnp.int32),
            pltpu.VMEM((CH, NHID), jnp.float32),
            pltpu.VMEM_SHARED((NPAD, NHID), jnp.float32),
            pltpu.SemaphoreType.DMA,
        ],
    )


def _edge_agg(x, srcp, dstp, zeros_stripe):
    return _edge_agg_kernel()(x, srcp, dstp, zeros_stripe)


def _encoder_body(x_ref, w_ref, b_ref, o_ref):
    h = jnp.dot(x_ref[...], w_ref[...],
                preferred_element_type=jnp.float32) + b_ref[...]
    rows = lax.broadcasted_iota(jnp.int32, (NPAD, 1), 0)
    o_ref[...] = jnp.where(rows < N_NODES, h, 0.0)


def _encoder(x, w, b):
    return pl.pallas_call(
        _encoder_body,
        out_shape=jax.ShapeDtypeStruct((NPAD, NHID), jnp.float32),
    )(x, w, b)


def _layer_mm_body(x_ref, agg_ref, w_ref, b_ref, p_ref, hl_ref, sc_ref):
    xa = x_ref[...] + agg_ref[0] + agg_ref[1]
    hl = jnp.dot(xa, w_ref[...],
                 preferred_element_type=jnp.float32) + b_ref[...]
    hl_ref[...] = hl
    p = p_ref[...]
    nrm = jnp.sqrt(jnp.sum(p * p))
    raw = jnp.dot(hl, p, preferred_element_type=jnp.float32)
    sc_ref[...] = jnp.tanh(raw / (nrm + 1e-8))


def _layer_mm(x, agg2, w, b, p_col):
    return pl.pallas_call(
        _layer_mm_body,
        out_shape=[
            jax.ShapeDtypeStruct((NPAD, NHID), jnp.float32),
            jax.ShapeDtypeStruct((NPAD, 1), jnp.float32),
        ],
    )(x, agg2, w, b, p_col)


def _order_key(bits):
    return jnp.where(bits >= 0, bits, bits ^ np.int32(0x7FFFFFFF))


def _select_body(k, L, hist2_ref, m2_ref, mo_ref, to_ref):
    cand = m2_ref[...] > 0.0
    sel = cand & (~cand)
    need = np.int32(k)
    for lvl in range(L):
        k2 = _order_key(lax.bitcast_convert_type(hist2_ref[lvl], jnp.int32))
        k2 = jnp.where(cand, k2, _I32MIN)
        curu = np.int32(0)
        for bit in range(31, -1, -1):
            mval = 1 << bit
            mi = np.int32(mval - 2 ** 32) if mval >= 2 ** 31 else np.int32(mval)
            cu = curu | mi
            cnt = jnp.sum((k2 >= (cu ^ _I32MIN)).astype(jnp.int32))
            curu = jnp.where(cnt >= need, cu, curu)
        thr = curu ^ _I32MIN
        gt = cand & (k2 > thr)
        need = need - jnp.sum(gt.astype(jnp.int32))
        sel = sel | gt
        cand = cand & (k2 == thr)
    idx2 = (lax.broadcasted_iota(jnp.int32, (R2D, 128), 0) * 128
            + lax.broadcasted_iota(jnp.int32, (R2D, 128), 1))
    c = np.int32(0)
    for bit in range(13, -1, -1):
        t = c + np.int32(1 << bit)
        f = jnp.sum((cand & (idx2 <= (t - 1))).astype(jnp.int32))
        c = jnp.where(f < need, t, c)
    climit = jnp.where(need > 0, c, np.int32(-1))
    sel = sel | (cand & (idx2 <= climit))
    mo_ref[...] = sel.astype(jnp.float32)
    to_ref[...] = jnp.where(sel, hist2_ref[0], 0.0)


def _select_search(k, hist2, m2d):
    L = hist2.shape[0]
    return pl.pallas_call(
        functools.partial(_select_body, k, L),
        out_shape=[
            jax.ShapeDtypeStruct((R2D, 128), jnp.float32),
            jax.ShapeDtypeStruct((R2D, 128), jnp.float32),
        ],
    )(hist2, m2d)


def _apply_body(hl_ref, t_ref, xo_ref):
    trow = t_ref[0]
    rr = lax.broadcasted_iota(jnp.int32, (128, 128), 0)
    cc = lax.broadcasted_iota(jnp.int32, (128, 128), 1)
    tmat = jnp.where(rr == cc, jnp.broadcast_to(trow, (128, 128)), 0.0)
    tcol = jnp.sum(tmat, axis=1, keepdims=True)
    xo_ref[...] = jnp.maximum(hl_ref[...] * tcol, 0.0)


def _apply(hl, t3):
    return pl.pallas_call(
        _apply_body,
        grid=(R2D,),
        in_specs=[
            pl.BlockSpec((128, NHID), lambda g: (g, 0)),
            pl.BlockSpec((1, 1, 128), lambda g: (g, 0, 0)),
        ],
        out_specs=pl.BlockSpec((128, NHID), lambda g: (g, 0)),
        out_shape=jax.ShapeDtypeStruct((NPAD, NHID), jnp.float32),
    )(hl, t3)


def _readout_body(batch_ref, x_ref, m_ref, xsin_ref, out_ref):
    g = pl.program_id(0)
    b2 = batch_ref[...]
    s = jnp.sum((b2 < g).astype(jnp.int32))
    e = jnp.sum((b2 <= g).astype(jnp.int32))
    neg = np.float32(-np.inf)

    def body(i, car):
        sm, mx, ct = car
        r0 = s + i * 8
        xr = x_ref[pl.ds(r0, 8), :]
        mr = m_ref[pl.ds(r0, 8), :]
        rid = r0 + lax.broadcasted_iota(jnp.int32, (8, 1), 0)
        inr = rid < e
        sm = sm + jnp.where(inr, xr, 0.0)
        livemask = inr & (mr > 0.0)
        mx = jnp.maximum(mx, jnp.where(livemask, xr, neg))
        ct = ct + jnp.sum(jnp.where(livemask, 1.0, 0.0))
        return sm, mx, ct

    init = (jnp.zeros((8, NHID), jnp.float32),
            jnp.full((8, NHID), neg, jnp.float32), np.float32(0.0))
    sm, mx, ct = lax.fori_loop(0, (e - s + 7) // 8, body, init)
    xsum = jnp.sum(sm, axis=0, keepdims=True)
    xmax = jnp.max(mx, axis=0, keepdims=True)
    xmax = jnp.where(xmax > neg, xmax, 0.0)
    xmean = xsum / jnp.maximum(ct, 1.0)
    row = jnp.concatenate([xsum, xmean, xmax], axis=1)
    out_ref[...] = xsin_ref[...] + row[None]


def _readout(batch2d, x, m_col, xs_in):
    return pl.pallas_call(
        _readout_body,
        grid=(G,),
        in_specs=[
            pl.BlockSpec((R2D, 128), lambda g: (0, 0)),
            pl.BlockSpec((NPAD, NHID), lambda g: (0, 0)),
            pl.BlockSpec((NPAD, 1), lambda g: (0, 0)),
            pl.BlockSpec((1, 1, 3 * NHID), lambda g: (g, 0, 0)),
        ],
        out_specs=pl.BlockSpec((1, 1, 3 * NHID), lambda g: (g, 0, 0)),
        out_shape=jax.ShapeDtypeStruct((G, 1, 3 * NHID), jnp.float32),
    )(batch2d, x, m_col, xs_in)


def _mlp_body(a_ref, b_ref, w1_ref, b1_ref, w2_ref, b2_ref, w3_ref, b3_ref,
              o_ref):
    feat = jnp.concatenate([a_ref[...], b_ref[...]], axis=1)
    h1 = jnp.maximum(jnp.dot(feat, w1_ref[...],
                             preferred_element_type=jnp.float32)
                     + b1_ref[...], 0.0)
    h2 = jnp.maximum(jnp.dot(h1, w2_ref[...],
                             preferred_element_type=jnp.float32)
                     + b2_ref[...], 0.0)
    o_ref[...] = jnp.dot(h2, w3_ref[...],
                         preferred_element_type=jnp.float32) + b3_ref[...]


def _mlp(xs0, xs1, W1, b1, W2, b2, W3, b3):
    return pl.pallas_call(
        _mlp_body,
        out_shape=jax.ShapeDtypeStruct((G, NTASKS), jnp.float32),
    )(xs0, xs1, W1, b1.reshape(1, -1), W2, b2.reshape(1, -1),
      W3, b3.reshape(1, -1))


def kernel(x, edge_index, batch, W_enc, b_enc, Ws, bs, ps, W1, b1, W2, b2,
           W3, b3):
    xp = jnp.zeros((NPAD, NHID), jnp.float32).at[:N_NODES].set(x)
    pad_e = jnp.full((EPAD - N_EDGES,), N_NODES, jnp.int32)
    srcp = jnp.concatenate([edge_index[0], pad_e]).reshape(NWORK, CHUNKS, CH)
    dstp = jnp.concatenate([edge_index[1], pad_e]).reshape(NWORK, CHUNKS, CH)
    batch2d = jnp.concatenate(
        [batch, jnp.full((NPAD - N_NODES,), G, jnp.int32)]).reshape(R2D, 128)
    m_col = (jnp.arange(NPAD, dtype=jnp.int32)
             < N_NODES).astype(jnp.float32).reshape(NPAD, 1)
    zeros_stripe = jnp.zeros((STRIPE, NHID), jnp.float32)

    xcur = _encoder(xp, W_enc, b_enc.reshape(1, NHID))
    xs = [jnp.zeros((G, 1, 3 * NHID), jnp.float32) for _ in range(HEADS)]
    n_live = N_NODES
    m2d = m_col.reshape(R2D, 128)
    hist = []
    for t in range(HEADS * LAYERS):
        l = t % LAYERS
        k = max(1, int(math.ceil(0.5 * n_live)))
        n_live = k
        agg2 = _edge_agg(xcur, srcp, dstp, zeros_stripe)
        hl, sc_col = _layer_mm(xcur, agg2, Ws[l], bs[l].reshape(1, NHID),
                               ps[l].reshape(NHID, 1))
        hist.insert(0, sc_col.reshape(R2D, 128))
        m2d, t2d = _select_search(k, jnp.stack(hist), m2d)
        xcur = _apply(hl, t2d.reshape(R2D, 1, 128))
        xs[t // LAYERS] = _readout(batch2d, xcur, m2d.reshape(NPAD, 1),
                                   xs[t // LAYERS])
    return _mlp(xs[0].reshape(G, 3 * NHID), xs[1].reshape(G, 3 * NHID),
                W1, b1, W2, b2, W3, b3)


# speedup vs baseline: 7.5982x; 1.0651x over previous
import functools
import math

import jax
import jax.numpy as jnp
import numpy as np
from jax import lax
from jax.experimental import pallas as pl
from jax.experimental.pallas import tpu as pltpu
from jax.experimental.pallas import tpu_sc as plsc

NHID = 128
HEADS = 2
LAYERS = 2
G = 64
NTASKS = 10
N_NODES = 10000
N_EDGES = 320000

NPAD = 10112
R2D = NPAD // 128
CH = 80
NWORK = 32
NSUB = 16
CHUNKS = 128
EPAD = NWORK * CHUNKS * CH
STRIPE = NPAD // NSUB
_I32MIN = np.int32(-(2 ** 31))


def _edge_agg_body(x_hbm, src_hbm, dst_hbm, zero_hbm, out_hbm,
                   src_v, dst_v, rows0_v, rows1_v, acc_sh, sem0, sem1):
    c = lax.axis_index("c")
    s = lax.axis_index("s")
    w = c * NSUB + s
    pltpu.sync_copy(zero_hbm, acc_sh.at[pl.ds(s * STRIPE, STRIPE)])
    plsc.subcore_barrier()

    def body(i, carry):
        j0 = i * 2
        j1 = j0 + 1
        h0 = pltpu.async_copy(x_hbm.at[src_v.at[j0]], rows0_v, sem0)
        h1 = pltpu.async_copy(x_hbm.at[src_v.at[j1]], rows1_v, sem1)
        h0.wait()
        pltpu.sync_copy(rows0_v, acc_sh.at[dst_v.at[j0]], add=True)
        h1.wait()
        pltpu.sync_copy(rows1_v, acc_sh.at[dst_v.at[j1]], add=True)
        return carry

    for phase in range(2):
        pltpu.sync_copy(src_hbm.at[w, pl.ds(phase * (CHUNKS // 2),
                                            CHUNKS // 2)], src_v)
        pltpu.sync_copy(dst_hbm.at[w, pl.ds(phase * (CHUNKS // 2),
                                            CHUNKS // 2)], dst_v)
        lax.fori_loop(0, CHUNKS // 4, body, 0)
    plsc.subcore_barrier()
    pltpu.sync_copy(acc_sh.at[pl.ds(s * STRIPE, STRIPE)],
                    out_hbm.at[c, pl.ds(s * STRIPE, STRIPE)])


@functools.cache
def _edge_agg_kernel():
    return pl.kernel(
        _edge_agg_body,
        mesh=plsc.VectorSubcoreMesh(core_axis_name="c", subcore_axis_name="s"),
        out_type=jax.ShapeDtypeStruct((2, NPAD, NHID), jnp.float32),
        scratch_types=[
            pltpu.VMEM((CHUNKS // 2, CH), jnp.int32),
            pltpu.VMEM((CHUNKS // 2, CH), jnp.int32),
            pltpu.VMEM((CH, NHID), jnp.float32),
            pltpu.VMEM((CH, NHID), jnp.float32),
            pltpu.VMEM_SHARED((NPAD, NHID), jnp.float32),
            pltpu.SemaphoreType.DMA,
            pltpu.SemaphoreType.DMA,
        ],
    )


def _edge_agg(x, srcp, dstp, zeros_stripe):
    return _edge_agg_kernel()(x, srcp, dstp, zeros_stripe)


def _encoder_body(x_ref, w_ref, b_ref, o_ref):
    h = jnp.dot(x_ref[...], w_ref[...],
                preferred_element_type=jnp.float32) + b_ref[...]
    rows = lax.broadcasted_iota(jnp.int32, (NPAD, 1), 0)
    o_ref[...] = jnp.where(rows < N_NODES, h, 0.0)


def _encoder(x, w, b):
    return pl.pallas_call(
        _encoder_body,
        out_shape=jax.ShapeDtypeStruct((NPAD, NHID), jnp.float32),
    )(x, w, b)


def _layer_mm_body(x_ref, agg_ref, w_ref, b_ref, p_ref, hl_ref, sc_ref):
    xa = x_ref[...] + agg_ref[0] + agg_ref[1]
    hl = jnp.dot(xa, w_ref[...],
                 preferred_element_type=jnp.float32) + b_ref[...]
    hl_ref[...] = hl
    p = p_ref[...]
    nrm = jnp.sqrt(jnp.sum(p * p))
    raw = jnp.dot(hl, p, preferred_element_type=jnp.float32)
    sc_ref[...] = jnp.tanh(raw / (nrm + 1e-8))


def _layer_mm(x, agg2, w, b, p_col):
    return pl.pallas_call(
        _layer_mm_body,
        out_shape=[
            jax.ShapeDtypeStruct((NPAD, NHID), jnp.float32),
            jax.ShapeDtypeStruct((NPAD, 1), jnp.float32),
        ],
    )(x, agg2, w, b, p_col)


def _order_key(bits):
    return jnp.where(bits >= 0, bits, bits ^ np.int32(0x7FFFFFFF))


def _select_body(k, L, hist2_ref, m2_ref, mo_ref, to_ref):
    cand = m2_ref[...] > 0.0
    sel = cand & (~cand)
    need = np.int32(k)
    for lvl in range(L):
        k2 = _order_key(lax.bitcast_convert_type(hist2_ref[lvl], jnp.int32))
        k2 = jnp.where(cand, k2, _I32MIN)
        curu = np.int32(0)
        for bit in range(31, -1, -1):
            mval = 1 << bit
            mi = np.int32(mval - 2 ** 32) if mval >= 2 ** 31 else np.int32(mval)
            cu = curu | mi
            cnt = jnp.sum((k2 >= (cu ^ _I32MIN)).astype(jnp.int32))
            curu = jnp.where(cnt >= need, cu, curu)
        thr = curu ^ _I32MIN
        gt = cand & (k2 > thr)
        need = need - jnp.sum(gt.astype(jnp.int32))
        sel = sel | gt
        cand = cand & (k2 == thr)
    idx2 = (lax.broadcasted_iota(jnp.int32, (R2D, 128), 0) * 128
            + lax.broadcasted_iota(jnp.int32, (R2D, 128), 1))
    c = np.int32(0)
    for bit in range(13, -1, -1):
        t = c + np.int32(1 << bit)
        f = jnp.sum((cand & (idx2 <= (t - 1))).astype(jnp.int32))
        c = jnp.where(f < need, t, c)
    climit = jnp.where(need > 0, c, np.int32(-1))
    sel = sel | (cand & (idx2 <= climit))
    mo_ref[...] = sel.astype(jnp.float32)
    to_ref[...] = jnp.where(sel, hist2_ref[0], 0.0)


def _select_search(k, hist2, m2d):
    L = hist2.shape[0]
    return pl.pallas_call(
        functools.partial(_select_body, k, L),
        out_shape=[
            jax.ShapeDtypeStruct((R2D, 128), jnp.float32),
            jax.ShapeDtypeStruct((R2D, 128), jnp.float32),
        ],
    )(hist2, m2d)


def _apply_body(hl_ref, t_ref, xo_ref):
    trow = t_ref[0]
    rr = lax.broadcasted_iota(jnp.int32, (128, 128), 0)
    cc = lax.broadcasted_iota(jnp.int32, (128, 128), 1)
    tmat = jnp.where(rr == cc, jnp.broadcast_to(trow, (128, 128)), 0.0)
    tcol = jnp.sum(tmat, axis=1, keepdims=True)
    xo_ref[...] = jnp.maximum(hl_ref[...] * tcol, 0.0)


def _apply(hl, t3):
    return pl.pallas_call(
        _apply_body,
        grid=(R2D,),
        in_specs=[
            pl.BlockSpec((128, NHID), lambda g: (g, 0)),
            pl.BlockSpec((1, 1, 128), lambda g: (g, 0, 0)),
        ],
        out_specs=pl.BlockSpec((128, NHID), lambda g: (g, 0)),
        out_shape=jax.ShapeDtypeStruct((NPAD, NHID), jnp.float32),
    )(hl, t3)


def _readout_body(batch_ref, x_ref, m_ref, xsin_ref, out_ref):
    g = pl.program_id(0)
    b2 = batch_ref[...]
    s = jnp.sum((b2 < g).astype(jnp.int32))
    e = jnp.sum((b2 <= g).astype(jnp.int32))
    neg = np.float32(-np.inf)

    def body(i, car):
        sm, mx, ct = car
        r0 = s + i * 8
        xr = x_ref[pl.ds(r0, 8), :]
        mr = m_ref[pl.ds(r0, 8), :]
        rid = r0 + lax.broadcasted_iota(jnp.int32, (8, 1), 0)
        inr = rid < e
        sm = sm + jnp.where(inr, xr, 0.0)
        livemask = inr & (mr > 0.0)
        mx = jnp.maximum(mx, jnp.where(livemask, xr, neg))
        ct = ct + jnp.sum(jnp.where(livemask, 1.0, 0.0))
        return sm, mx, ct

    init = (jnp.zeros((8, NHID), jnp.float32),
            jnp.full((8, NHID), neg, jnp.float32), np.float32(0.0))
    sm, mx, ct = lax.fori_loop(0, (e - s + 7) // 8, body, init)
    xsum = jnp.sum(sm, axis=0, keepdims=True)
    xmax = jnp.max(mx, axis=0, keepdims=True)
    xmax = jnp.where(xmax > neg, xmax, 0.0)
    xmean = xsum / jnp.maximum(ct, 1.0)
    row = jnp.concatenate([xsum, xmean, xmax], axis=1)
    out_ref[...] = xsin_ref[...] + row[None]


def _readout(batch2d, x, m_col, xs_in):
    return pl.pallas_call(
        _readout_body,
        grid=(G,),
        in_specs=[
            pl.BlockSpec((R2D, 128), lambda g: (0, 0)),
            pl.BlockSpec((NPAD, NHID), lambda g: (0, 0)),
            pl.BlockSpec((NPAD, 1), lambda g: (0, 0)),
            pl.BlockSpec((1, 1, 3 * NHID), lambda g: (g, 0, 0)),
        ],
        out_specs=pl.BlockSpec((1, 1, 3 * NHID), lambda g: (g, 0, 0)),
        out_shape=jax.ShapeDtypeStruct((G, 1, 3 * NHID), jnp.float32),
    )(batch2d, x, m_col, xs_in)


def _mlp_body(a_ref, b_ref, w1_ref, b1_ref, w2_ref, b2_ref, w3_ref, b3_ref,
              o_ref):
    feat = jnp.concatenate([a_ref[...], b_ref[...]], axis=1)
    h1 = jnp.maximum(jnp.dot(feat, w1_ref[...],
                             preferred_element_type=jnp.float32)
                     + b1_ref[...], 0.0)
    h2 = jnp.maximum(jnp.dot(h1, w2_ref[...],
                             preferred_element_type=jnp.float32)
                     + b2_ref[...], 0.0)
    o_ref[...] = jnp.dot(h2, w3_ref[...],
                         preferred_element_type=jnp.float32) + b3_ref[...]


def _mlp(xs0, xs1, W1, b1, W2, b2, W3, b3):
    return pl.pallas_call(
        _mlp_body,
        out_shape=jax.ShapeDtypeStruct((G, NTASKS), jnp.float32),
    )(xs0, xs1, W1, b1.reshape(1, -1), W2, b2.reshape(1, -1),
      W3, b3.reshape(1, -1))


def kernel(x, edge_index, batch, W_enc, b_enc, Ws, bs, ps, W1, b1, W2, b2,
           W3, b3):
    xp = jnp.zeros((NPAD, NHID), jnp.float32).at[:N_NODES].set(x)
    pad_e = jnp.full((EPAD - N_EDGES,), N_NODES, jnp.int32)
    srcp = jnp.concatenate([edge_index[0], pad_e]).reshape(NWORK, CHUNKS, CH)
    dstp = jnp.concatenate([edge_index[1], pad_e]).reshape(NWORK, CHUNKS, CH)
    batch2d = jnp.concatenate(
        [batch, jnp.full((NPAD - N_NODES,), G, jnp.int32)]).reshape(R2D, 128)
    m_col = (jnp.arange(NPAD, dtype=jnp.int32)
             < N_NODES).astype(jnp.float32).reshape(NPAD, 1)
    zeros_stripe = jnp.zeros((STRIPE, NHID), jnp.float32)

    xcur = _encoder(xp, W_enc, b_enc.reshape(1, NHID))
    xs = [jnp.zeros((G, 1, 3 * NHID), jnp.float32) for _ in range(HEADS)]
    n_live = N_NODES
    m2d = m_col.reshape(R2D, 128)
    hist = []
    for t in range(HEADS * LAYERS):
        l = t % LAYERS
        k = max(1, int(math.ceil(0.5 * n_live)))
        n_live = k
        agg2 = _edge_agg(xcur, srcp, dstp, zeros_stripe)
        hl, sc_col = _layer_mm(xcur, agg2, Ws[l], bs[l].reshape(1, NHID),
                               ps[l].reshape(NHID, 1))
        hist.insert(0, sc_col.reshape(R2D, 128))
        m2d, t2d = _select_search(k, jnp.stack(hist), m2d)
        xcur = _apply(hl, t2d.reshape(R2D, 1, 128))
        xs[t // LAYERS] = _readout(batch2d, xcur, m2d.reshape(NPAD, 1),
                                   xs[t // LAYERS])
    return _mlp(xs[0].reshape(G, 3 * NHID), xs[1].reshape(G, 3 * NHID),
                W1, b1, W2, b2, W3, b3)
